# baseline (speedup 1.0000x reference)
import functools

import jax
import jax.numpy as jnp
from jax import lax
from jax.experimental import pallas as pl
from jax.experimental.pallas import tpu as pltpu
from jax.experimental.pallas import tpu_sc as plsc

N = 1000000
D = 64
D2 = 2 * D
B = 16384
BH = B // 2
K = 20
POS_MARGIN = 2.0
NEG_MARGIN = 1.0
ZERO_MARGIN = 0.5

NC = 2
NS = 16
NW = NC * NS

CH = 256


def _sc_gather(H, T, h, t, nt):
    Bx = h.shape[0]
    BK = nt.shape[0]
    bw = Bx // NW
    nw = BK // NW
    mesh = plsc.VectorSubcoreMesh(
        core_axis_name="c", subcore_axis_name="s", num_cores=NC, num_subcores=NS
    )

    @functools.partial(
        pl.kernel,
        out_type=(
            jax.ShapeDtypeStruct((Bx, D), jnp.float32),
            jax.ShapeDtypeStruct((Bx, D), jnp.float32),
            jax.ShapeDtypeStruct((BK, D), jnp.float32),
        ),
        mesh=mesh,
        scratch_types=[
            pltpu.VMEM((CH,), jnp.int32),
            pltpu.VMEM((CH, D), jnp.float32),
            pltpu.SemaphoreType.DMA,
        ],
        compiler_params=pltpu.CompilerParams(use_tc_tiling_on_sc=False),
    )
    def k(H_hbm, T_hbm, h_hbm, t_hbm, nt_hbm, hr_hbm, tr_hbm, ntr_hbm,
          idx_v, rows_v, sem):
        wid = lax.axis_index("s") * NC + lax.axis_index("c")
        base = wid * bw
        pltpu.sync_copy(h_hbm.at[pl.ds(base, bw)], idx_v)
        pltpu.async_copy(H_hbm.at[idx_v], rows_v, sem).wait()
        pltpu.sync_copy(rows_v, hr_hbm.at[pl.ds(base, bw)])
        pltpu.sync_copy(t_hbm.at[pl.ds(base, bw)], idx_v)
        pltpu.async_copy(T_hbm.at[idx_v], rows_v, sem).wait()
        pltpu.sync_copy(rows_v, tr_hbm.at[pl.ds(base, bw)])

        nbase = wid * nw

        @pl.loop(0, nw, step=CH)
        def _(off):
            pltpu.sync_copy(nt_hbm.at[pl.ds(nbase + off, CH)], idx_v)
            pltpu.async_copy(T_hbm.at[idx_v], rows_v, sem).wait()
            pltpu.sync_copy(rows_v, ntr_hbm.at[pl.ds(nbase + off, CH)])

    return k(H, T, h, t, nt)


BBH = 512


def _rsel(ri, x0, x1, x2):
    return jnp.where(ri == 0, x0, jnp.where(ri == 1, x1, x2))


def _tc_loss_kernel(h_ref, t_ref, nt_ref, se_ref, so_ref, re_ref, ro_ref,
                    nre_ref, nro_ref, Rd_ref, out_ref):
    h2 = h_ref[...]
    t2 = t_ref[...]
    se = se_ref[...]
    so = so_ref[...]
    rie = re_ref[...]
    rio = ro_ref[...]
    lane = lax.broadcasted_iota(jnp.int32, (BBH, D2), 1)
    msk = lane < D
    sv = jnp.where(msk, se, so)
    Rrows = [Rd_ref[j:j + 1, :] for j in range(3)]
    r_emb = jnp.where(msk, _rsel(rie, *Rrows), _rsel(rio, *Rrows))
    wl = lax.broadcasted_iota(jnp.int32, (D2, 2), 0)
    wc = lax.broadcasted_iota(jnp.int32, (D2, 2), 1)
    W = (((wl < D) & (wc == 0)) | ((wl >= D) & (wc == 1))).astype(jnp.float32)
    dpos = jnp.abs(h2 + sv * r_emb - t2)
    psums = lax.dot_general(dpos, W, (((1,), (0,)), ((), ())),
                            preferred_element_type=jnp.float32)
    pos_e = psums[:, 0:1]
    pos_o = psums[:, 1:2]
    hsd = [h2 + sv * Rrows[j] for j in range(3)]
    acc = jnp.float32(0.0)
    for k in range(K):
        ntk = nt_ref[k]
        dcat = jnp.concatenate(
            [jnp.abs(hsd[j] - ntk) for j in range(3)], axis=0)
        sums = lax.dot_general(dcat, W, (((1,), (0,)), ((), ())),
                               preferred_element_type=jnp.float32)
        nre = nre_ref[:, k:k + 1]
        nro = nro_ref[:, k:k + 1]
        neg_e = _rsel(nre, sums[0:BBH, 0:1], sums[BBH:2 * BBH, 0:1],
                      sums[2 * BBH:3 * BBH, 0:1])
        neg_o = _rsel(nro, sums[0:BBH, 1:2], sums[BBH:2 * BBH, 1:2],
                      sums[2 * BBH:3 * BBH, 1:2])
        m_e = _rsel(nre, NEG_MARGIN, POS_MARGIN, ZERO_MARGIN)
        m_o = _rsel(nro, NEG_MARGIN, POS_MARGIN, ZERO_MARGIN)
        acc += (jnp.sum(jnp.maximum(0.0, m_e + pos_e - neg_e))
                + jnp.sum(jnp.maximum(0.0, m_o + pos_o - neg_o)))

    @pl.when(pl.program_id(0) == 0)
    def _():
        out_ref[...] = jnp.zeros_like(out_ref)

    out_ref[...] = out_ref[...] + acc


def _tc_loss(h2, t2, nt2, s_e, s_o, r_e, r_o, nr_e, nr_o, R_dup):
    grid = (h2.shape[0] // BBH,)
    return pl.pallas_call(
        _tc_loss_kernel,
        grid=grid,
        in_specs=[
            pl.BlockSpec((BBH, D2), lambda i: (i, 0)),
            pl.BlockSpec((BBH, D2), lambda i: (i, 0)),
            pl.BlockSpec((K, BBH, D2), lambda i: (0, i, 0)),
            pl.BlockSpec((BBH, 1), lambda i: (i, 0)),
            pl.BlockSpec((BBH, 1), lambda i: (i, 0)),
            pl.BlockSpec((BBH, 1), lambda i: (i, 0)),
            pl.BlockSpec((BBH, 1), lambda i: (i, 0)),
            pl.BlockSpec((BBH, K), lambda i: (i, 0)),
            pl.BlockSpec((BBH, K), lambda i: (i, 0)),
            pl.BlockSpec((8, D2), lambda i: (0, 0)),
        ],
        out_specs=pl.BlockSpec((1, 1), lambda i: (0, 0)),
        out_shape=jax.ShapeDtypeStruct((1, 1), jnp.float32),
    )(h2, t2, nt2, s_e, s_o, r_e, r_o, nr_e, nr_o, R_dup)


def _half_loss(h, r, t, sign, negs_r, negs_t, H, R_dup, T):
    Bx = h.shape[0]
    BxH = Bx // 2
    nt_kflat = negs_t.T.reshape(Bx * K)
    hrows, trows, ntrows = _sc_gather(H, T, h, t, nt_kflat)
    h2 = hrows.reshape(BxH, D2)
    t2 = trows.reshape(BxH, D2)
    nt2 = ntrows.reshape(K, BxH, D2)
    sign_f = sign.astype(jnp.float32)
    s_e = sign_f[0::2].reshape(BxH, 1)
    s_o = sign_f[1::2].reshape(BxH, 1)
    r_e = r[0::2].reshape(BxH, 1)
    r_o = r[1::2].reshape(BxH, 1)
    nr_e = negs_r[0::2, :]
    nr_o = negs_r[1::2, :]
    return _tc_loss(h2, t2, nt2, s_e, s_o, r_e, r_o, nr_e, nr_o, R_dup)


def kernel(h, r, t, sign, negs_r, negs_t, H, R, T):
    h = h.astype(jnp.int32)
    t = t.astype(jnp.int32)
    r = r.astype(jnp.int32)
    nr = negs_r.astype(jnp.int32)
    nt = negs_t.astype(jnp.int32)
    R_dup = (jnp.zeros((8, D2), jnp.float32)
             .at[:3, :D].set(R).at[:3, D:].set(R))
    M = B // 2
    out0 = _half_loss(h[:M], r[:M], t[:M], sign[:M], nr[:M], nt[:M],
                      H, R_dup, T)
    out1 = _half_loss(h[M:], r[M:], t[M:], sign[M:], nr[M:], nt[M:],
                      H, R_dup, T)
    return (out0 + out1).reshape(())


# speedup vs baseline: 1.0209x; 1.0209x over previous
import functools

import jax
import jax.numpy as jnp
from jax import lax
from jax.experimental import pallas as pl
from jax.experimental.pallas import tpu as pltpu
from jax.experimental.pallas import tpu_sc as plsc

N = 1000000
D = 64
D2 = 2 * D
B = 16384
BH = B // 2
K = 20
POS_MARGIN = 2.0
NEG_MARGIN = 1.0
ZERO_MARGIN = 0.5

NC = 2
NS = 16
NW = NC * NS

CH = 512


def _sc_gather(H, T, h, t, nt):
    BK = nt.shape[0]
    bw = B // NW
    nw = BK // NW
    mesh = plsc.VectorSubcoreMesh(
        core_axis_name="c", subcore_axis_name="s", num_cores=NC, num_subcores=NS
    )

    @functools.partial(
        pl.kernel,
        out_type=(
            jax.ShapeDtypeStruct((B, D), jnp.float32),
            jax.ShapeDtypeStruct((B, D), jnp.float32),
            jax.ShapeDtypeStruct((BK, D), jnp.float32),
        ),
        mesh=mesh,
        scratch_types=[
            pltpu.VMEM((CH,), jnp.int32),
            pltpu.VMEM((CH, D), jnp.float32),
            pltpu.SemaphoreType.DMA,
        ],
        compiler_params=pltpu.CompilerParams(use_tc_tiling_on_sc=False),
    )
    def k(H_hbm, T_hbm, h_hbm, t_hbm, nt_hbm, hr_hbm, tr_hbm, ntr_hbm,
          idx_v, rows_v, sem):
        wid = lax.axis_index("s") * NC + lax.axis_index("c")
        base = wid * bw
        pltpu.sync_copy(h_hbm.at[pl.ds(base, bw)], idx_v)
        pltpu.async_copy(H_hbm.at[idx_v], rows_v, sem).wait()
        pltpu.sync_copy(rows_v, hr_hbm.at[pl.ds(base, bw)])
        pltpu.sync_copy(t_hbm.at[pl.ds(base, bw)], idx_v)
        pltpu.async_copy(T_hbm.at[idx_v], rows_v, sem).wait()
        pltpu.sync_copy(rows_v, tr_hbm.at[pl.ds(base, bw)])

        nbase = wid * nw

        @pl.loop(0, nw, step=CH)
        def _(off):
            pltpu.sync_copy(nt_hbm.at[pl.ds(nbase + off, CH)], idx_v)
            pltpu.async_copy(T_hbm.at[idx_v], rows_v, sem).wait()
            pltpu.sync_copy(rows_v, ntr_hbm.at[pl.ds(nbase + off, CH)])

    return k(H, T, h, t, nt)


BBH = 1024


def _rsel(ri, x0, x1, x2):
    return jnp.where(ri == 0, x0, jnp.where(ri == 1, x1, x2))


def _tc_loss_kernel(h_ref, t_ref, nt_ref, se_ref, so_ref, re_ref, ro_ref,
                    nre_ref, nro_ref, Rd_ref, out_ref):
    h2 = h_ref[...]
    t2 = t_ref[...]
    se = se_ref[...]
    so = so_ref[...]
    rie = re_ref[...]
    rio = ro_ref[...]
    lane = lax.broadcasted_iota(jnp.int32, (BBH, D2), 1)
    msk = lane < D
    sv = jnp.where(msk, se, so)
    Rrows = [Rd_ref[j:j + 1, :] for j in range(3)]
    r_emb = jnp.where(msk, _rsel(rie, *Rrows), _rsel(rio, *Rrows))
    wl = lax.broadcasted_iota(jnp.int32, (D2, 2), 0)
    wc = lax.broadcasted_iota(jnp.int32, (D2, 2), 1)
    W = (((wl < D) & (wc == 0)) | ((wl >= D) & (wc == 1))).astype(jnp.float32)
    dpos = jnp.abs(h2 + sv * r_emb - t2)
    psums = lax.dot_general(dpos, W, (((1,), (0,)), ((), ())),
                            preferred_element_type=jnp.float32)
    pos_e = psums[:, 0:1]
    pos_o = psums[:, 1:2]
    hsd = [h2 + sv * Rrows[j] for j in range(3)]
    acc = jnp.float32(0.0)
    for k in range(K):
        ntk = nt_ref[k]
        dcat = jnp.concatenate(
            [jnp.abs(hsd[j] - ntk) for j in range(3)], axis=0)
        sums = lax.dot_general(dcat, W, (((1,), (0,)), ((), ())),
                               preferred_element_type=jnp.float32)
        nre = nre_ref[:, k:k + 1]
        nro = nro_ref[:, k:k + 1]
        neg_e = _rsel(nre, sums[0:BBH, 0:1], sums[BBH:2 * BBH, 0:1],
                      sums[2 * BBH:3 * BBH, 0:1])
        neg_o = _rsel(nro, sums[0:BBH, 1:2], sums[BBH:2 * BBH, 1:2],
                      sums[2 * BBH:3 * BBH, 1:2])
        m_e = _rsel(nre, NEG_MARGIN, POS_MARGIN, ZERO_MARGIN)
        m_o = _rsel(nro, NEG_MARGIN, POS_MARGIN, ZERO_MARGIN)
        acc += (jnp.sum(jnp.maximum(0.0, m_e + pos_e - neg_e))
                + jnp.sum(jnp.maximum(0.0, m_o + pos_o - neg_o)))

    @pl.when(pl.program_id(0) == 0)
    def _():
        out_ref[...] = jnp.zeros_like(out_ref)

    out_ref[...] = out_ref[...] + acc


def _tc_loss(h2, t2, nt2, s_e, s_o, r_e, r_o, nr_e, nr_o, R_dup):
    grid = (BH // BBH,)
    return pl.pallas_call(
        _tc_loss_kernel,
        grid=grid,
        in_specs=[
            pl.BlockSpec((BBH, D2), lambda i: (i, 0)),
            pl.BlockSpec((BBH, D2), lambda i: (i, 0)),
            pl.BlockSpec((K, BBH, D2), lambda i: (0, i, 0)),
            pl.BlockSpec((BBH, 1), lambda i: (i, 0)),
            pl.BlockSpec((BBH, 1), lambda i: (i, 0)),
            pl.BlockSpec((BBH, 1), lambda i: (i, 0)),
            pl.BlockSpec((BBH, 1), lambda i: (i, 0)),
            pl.BlockSpec((BBH, K), lambda i: (i, 0)),
            pl.BlockSpec((BBH, K), lambda i: (i, 0)),
            pl.BlockSpec((8, D2), lambda i: (0, 0)),
        ],
        out_specs=pl.BlockSpec((1, 1), lambda i: (0, 0)),
        out_shape=jax.ShapeDtypeStruct((1, 1), jnp.float32),
    )(h2, t2, nt2, s_e, s_o, r_e, r_o, nr_e, nr_o, R_dup)


def kernel(h, r, t, sign, negs_r, negs_t, H, R, T):
    h = h.astype(jnp.int32)
    t = t.astype(jnp.int32)
    nt_kflat = negs_t.astype(jnp.int32).T.reshape(B * K)
    hrows, trows, ntrows = _sc_gather(H, T, h, t, nt_kflat)
    h2 = hrows.reshape(BH, D2)
    t2 = trows.reshape(BH, D2)
    nt2 = ntrows.reshape(K, BH, D2)
    sign_f = sign.astype(jnp.float32)
    s_e = sign_f[0::2].reshape(BH, 1)
    s_o = sign_f[1::2].reshape(BH, 1)
    r_i = r.astype(jnp.int32)
    r_e = r_i[0::2].reshape(BH, 1)
    r_o = r_i[1::2].reshape(BH, 1)
    nr = negs_r.astype(jnp.int32)
    nr_e = nr[0::2, :]
    nr_o = nr[1::2, :]
    R_dup = (jnp.zeros((8, D2), jnp.float32)
             .at[:3, :D].set(R).at[:3, D:].set(R))
    out = _tc_loss(h2, t2, nt2, s_e, s_o, r_e, r_o, nr_e, nr_o, R_dup)
    return out.reshape(())
